# baseline (speedup 1.0000x reference)
import jax
import jax.numpy as jnp
from jax import lax
from jax.experimental import pallas as pl
from jax.experimental.pallas import tpu as pltpu

_GAMMA = 2.0
_SC = 0.0001
_ALPHA = 0.25

_N = 20000
_C = 80
_G = 128
_BN = 4000
_NB = _N // _BN
_P = 256

_DN_STD = (((1,), (0,)), ((), ()))
_DN_TT = (((0,), (0,)), ((), ()))


def _mask_loss_kernel(oh2_ref, pk_ref, iib_ref,
                      out_ref, la, gsc, ab, conf, macc, negacc, coracc,
                      acc1, acc2, acc3):
    s = pl.program_id(0)
    i = pl.program_id(1)

    @pl.when((s == 0) & (i == 0))
    def _init():
        conf[...] = jnp.zeros_like(conf)
        macc[...] = jnp.zeros_like(macc)
        negacc[...] = jnp.zeros_like(negacc)
        coracc[...] = jnp.zeros_like(coracc)
        acc1[...] = jnp.zeros_like(acc1)
        acc2[...] = jnp.zeros_like(acc2)
        acc3[...] = jnp.zeros_like(acc3)

    @pl.when(s == 0)
    def _stage0():
        onesr = jnp.ones((1, _BN), jnp.float32)
        pk = pk_ref[...]
        lb = pk[:, 0:128]
        bp = jnp.clip(jax.nn.sigmoid(lb), _SC, 1.0 - _SC)
        onem = 1.0 - bp
        l1m = jnp.log(onem)
        fneg = -l1m * bp * bp
        negacc[...] += lax.dot_general(
            onesr, fneg, _DN_STD, preferred_element_type=jnp.float32)
        iib = iib_ref[...]
        iibf = iib.astype(jnp.float32)
        mask = iib > 0
        kk = lax.dot_general(iibf, jnp.ones((_G, 1), jnp.float32), _DN_STD,
                             preferred_element_type=jnp.float32)
        b = -fneg[:, 0:1]
        logp0 = jnp.clip(lb[:, 0:1] + l1m[:, 0:1],
                         jnp.log(_SC), jnp.log(1.0 - _SC))
        a = logp0 * onem[:, 0:1] * onem[:, 0:1]
        selb = jnp.where(kk > 0.0, b, 0.0)
        coracc[...] += lax.dot_general(
            onesr, selb, _DN_STD, preferred_element_type=jnp.float32)
        ab2 = jnp.concatenate([a, b], axis=1)
        ab[pl.ds(i * _BN, _BN), :] = ab2
        acc3[...] += lax.dot_general(
            ab2, iibf, _DN_TT, preferred_element_type=jnp.float32)
        ag = lax.dot_general(pk, oh2_ref[...], _DN_STD,
                             preferred_element_type=jnp.float32)
        align = ag[:, 0:128] * iibf
        gsc[pl.ds(i * _BN, _BN), :] = ag[:, 128:256]
        conf[...] = jnp.maximum(conf[...],
                                jnp.max(align, axis=0, keepdims=True))
        la[pl.ds(i * _BN, _BN), :] = jnp.where(
            mask, jnp.log(jnp.maximum(align, 1e-38)), 1.0)

    @pl.when(s == 1)
    def _stage1():
        lab = la[pl.ds(i * _BN, _BN), :]
        giou = gsc[pl.ds(i * _BN, _BN), :]
        maskb = lab < 0.5
        p = jnp.exp(conf[...] * lab)
        q = p * giou + _SC
        mq = jnp.where(maskb, q, 0.0)
        mq2 = mq * q
        macc[...] = jnp.maximum(macc[...],
                                jnp.max(mq, axis=0, keepdims=True))
        abb = ab[pl.ds(i * _BN, _BN), :]
        acc1[...] += lax.dot_general(
            abb, mq2, _DN_TT, preferred_element_type=jnp.float32)
        acc2[...] += lax.dot_general(
            abb, mq, _DN_TT, preferred_element_type=jnp.float32)

    @pl.when((s == 1) & (i == _NB - 1))
    def _final():
        rinv = 1.0 / jnp.maximum(macc[...], _SC)
        rinv2 = rinv * rinv
        posv = acc1[0:1, :] * rinv2
        bnegv = (acc3[1:2, :] - 2.0 * acc2[1:2, :] * rinv
                 + acc1[1:2, :] * rinv2)
        sneg = jnp.sum(negacc[:, 0:_C], axis=1, keepdims=True)
        spos = jnp.sum(posv, axis=1, keepdims=True)
        sbneg = jnp.sum(bnegv, axis=1, keepdims=True)
        total = ((1.0 - _ALPHA) * (sneg + coracc[...])
                 - _ALPHA * (spos + sbneg))
        out_ref[...] = jnp.broadcast_to(total, out_ref.shape)


def kernel(logits_pred, scores, iou_map, is_in_boxes, labels, num_pos_avg):
    packed = jnp.zeros((_N, _P), jnp.float32)
    packed = lax.dynamic_update_slice(packed, logits_pred, (0, 0))
    packed = lax.dynamic_update_slice(packed, scores, (0, _C))
    packed = lax.dynamic_update_slice(packed, iou_map, (0, 2 * _C))
    row = jax.lax.broadcasted_iota(jnp.int32, (_P, _G), 0)
    oh2 = jnp.concatenate(
        [(row == labels[None, :] + _C).astype(jnp.float32),
         (row == labels[None, :] + 2 * _C).astype(jnp.float32)],
        axis=1)
    out = pl.pallas_call(
        _mask_loss_kernel,
        grid=(2, _NB),
        in_specs=[
            pl.BlockSpec((_P, 2 * _G), lambda s, i: (0, 0)),
            pl.BlockSpec((_BN, _P),
                         lambda s, i: (jnp.where(s == 0, i, _NB - 1), 0)),
            pl.BlockSpec((_BN, _G),
                         lambda s, i: (jnp.where(s == 0, i, _NB - 1), 0)),
        ],
        out_specs=pl.BlockSpec((8, 128), lambda s, i: (0, 0)),
        out_shape=jax.ShapeDtypeStruct((8, 128), jnp.float32),
        scratch_shapes=[
            pltpu.VMEM((_N, _G), jnp.float32),
            pltpu.VMEM((_N, _G), jnp.float32),
            pltpu.VMEM((_N, 2), jnp.float32),
            pltpu.VMEM((1, _G), jnp.float32),
            pltpu.VMEM((1, _G), jnp.float32),
            pltpu.VMEM((1, 128), jnp.float32),
            pltpu.VMEM((1, 1), jnp.float32),
            pltpu.VMEM((2, _G), jnp.float32),
            pltpu.VMEM((2, _G), jnp.float32),
            pltpu.VMEM((2, _G), jnp.float32),
        ],
        compiler_params=pltpu.CompilerParams(
            dimension_semantics=("arbitrary", "arbitrary")),
    )(oh2, packed, is_in_boxes)
    return out[0, 0] / num_pos_avg


# speedup vs baseline: 3.9326x; 3.9326x over previous
import jax
import jax.numpy as jnp
from jax import lax
from jax.experimental import pallas as pl
from jax.experimental.pallas import tpu as pltpu

_GAMMA = 2.0
_SC = 0.0001
_ALPHA = 0.25

_N = 20000
_C = 80
_G = 128
_BN = 4000
_NB = _N // _BN

_DN_STD = (((1,), (0,)), ((), ()))
_DN_TT = (((0,), (0,)), ((), ()))


def _mask_loss_kernel(oh_ref, scores_ref, iib_ref, logits_ref, iou_ref,
                      out_ref, la, conf, macc, negacc, coracc,
                      acc1, acc2, acc3):
    s = pl.program_id(0)
    i = pl.program_id(1)

    @pl.when((s == 0) & (i == 0))
    def _init():
        conf[...] = jnp.zeros_like(conf)
        macc[...] = jnp.zeros_like(macc)
        negacc[...] = jnp.zeros_like(negacc)
        coracc[...] = jnp.zeros_like(coracc)
        acc1[...] = jnp.zeros_like(acc1)
        acc2[...] = jnp.zeros_like(acc2)
        acc3[...] = jnp.zeros_like(acc3)

    @pl.when(s == 0)
    def _stage0():
        iib = iib_ref[...]
        iibf = iib.astype(jnp.float32)
        mask = iib > 0
        align = lax.dot_general(scores_ref[...], oh_ref[...], _DN_STD,
                                preferred_element_type=jnp.float32) * iibf
        conf[...] = jnp.maximum(conf[...],
                                jnp.max(align, axis=0, keepdims=True))
        la[pl.ds(i * _BN, _BN), :] = jnp.where(
            mask, jnp.log(jnp.maximum(align, 1e-38)), 1.0)

    @pl.when(s == 1)
    def _stage1():
        onesr = jnp.ones((1, _BN), jnp.float32)
        lb = logits_ref[...]
        bp = jnp.clip(jax.nn.sigmoid(lb), _SC, 1.0 - _SC)
        onem = 1.0 - bp
        l1m = jnp.log(onem)
        fneg = -l1m * bp * bp
        negacc[...] += lax.dot_general(
            onesr, fneg, _DN_STD, preferred_element_type=jnp.float32)
        lab = la[pl.ds(i * _BN, _BN), :]
        maskb = lab < 0.5
        maskf = jnp.where(maskb, 1.0, 0.0)
        kk = lax.dot_general(maskf, jnp.ones((_G, 1), jnp.float32), _DN_STD,
                             preferred_element_type=jnp.float32)
        b = -fneg[:, 0:1]
        logp0 = jnp.clip(lb[:, 0:1] + l1m[:, 0:1],
                         jnp.log(_SC), jnp.log(1.0 - _SC))
        a = logp0 * onem[:, 0:1] * onem[:, 0:1]
        selb = jnp.where(kk > 0.0, b, 0.0)
        coracc[...] += lax.dot_general(
            onesr, selb, _DN_STD, preferred_element_type=jnp.float32)
        ab2 = jnp.concatenate([a, b], axis=1)
        giou = lax.dot_general(iou_ref[...], oh_ref[...], _DN_STD,
                               preferred_element_type=jnp.float32)
        p = jnp.exp(conf[...] * lab)
        q = p * giou + _SC
        mq = maskf * q
        mq2 = mq * q
        macc[...] = jnp.maximum(macc[...],
                                jnp.max(mq, axis=0, keepdims=True))
        acc1[...] += lax.dot_general(
            ab2, mq2, _DN_TT, preferred_element_type=jnp.float32)
        acc2[...] += lax.dot_general(
            ab2, mq, _DN_TT, preferred_element_type=jnp.float32)
        acc3[...] += lax.dot_general(
            ab2, maskf, _DN_TT, preferred_element_type=jnp.float32)

    @pl.when((s == 1) & (i == _NB - 1))
    def _final():
        rinv = 1.0 / jnp.maximum(macc[...], _SC)
        rinv2 = rinv * rinv
        posv = acc1[0:1, :] * rinv2
        bnegv = (acc3[1:2, :] - 2.0 * acc2[1:2, :] * rinv
                 + acc1[1:2, :] * rinv2)
        sneg = jnp.sum(negacc[...], axis=1, keepdims=True)
        spos = jnp.sum(posv, axis=1, keepdims=True)
        sbneg = jnp.sum(bnegv, axis=1, keepdims=True)
        total = ((1.0 - _ALPHA) * (sneg + coracc[...])
                 - _ALPHA * (spos + sbneg))
        out_ref[...] = jnp.broadcast_to(total, out_ref.shape)


def kernel(logits_pred, scores, iou_map, is_in_boxes, labels, num_pos_avg):
    oh = (labels[None, :] ==
          jax.lax.broadcasted_iota(jnp.int32, (_C, _G), 0)).astype(jnp.float32)
    out = pl.pallas_call(
        _mask_loss_kernel,
        grid=(2, _NB),
        in_specs=[
            pl.BlockSpec((_C, _G), lambda s, i: (0, 0)),
            pl.BlockSpec((_BN, _C),
                         lambda s, i: (jnp.where(s == 0, i, _NB - 1), 0)),
            pl.BlockSpec((_BN, _G),
                         lambda s, i: (jnp.where(s == 0, i, _NB - 1), 0)),
            pl.BlockSpec((_BN, _C), lambda s, i: (jnp.where(s == 1, i, 0), 0)),
            pl.BlockSpec((_BN, _C), lambda s, i: (jnp.where(s == 1, i, 0), 0)),
        ],
        out_specs=pl.BlockSpec((8, 128), lambda s, i: (0, 0)),
        out_shape=jax.ShapeDtypeStruct((8, 128), jnp.float32),
        scratch_shapes=[
            pltpu.VMEM((_N, _G), jnp.float32),
            pltpu.VMEM((1, _G), jnp.float32),
            pltpu.VMEM((1, _G), jnp.float32),
            pltpu.VMEM((1, _C), jnp.float32),
            pltpu.VMEM((1, 1), jnp.float32),
            pltpu.VMEM((2, _G), jnp.float32),
            pltpu.VMEM((2, _G), jnp.float32),
            pltpu.VMEM((2, _G), jnp.float32),
        ],
        compiler_params=pltpu.CompilerParams(
            dimension_semantics=("arbitrary", "arbitrary")),
    )(oh, scores, is_in_boxes, logits_pred, iou_map)
    return out[0, 0] / num_pos_avg


# speedup vs baseline: 4.0578x; 1.0318x over previous
import jax
import jax.numpy as jnp
from jax import lax
from jax.experimental import pallas as pl
from jax.experimental.pallas import tpu as pltpu

_GAMMA = 2.0
_SC = 0.0001
_ALPHA = 0.25

_N = 20000
_C = 80
_G = 128
_BN = 4000
_NB = _N // _BN

_DN_STD = (((1,), (0,)), ((), ()))
_DN_TT = (((0,), (0,)), ((), ()))


def _mask_loss_kernel(oh_ref, logits_ref, scores_ref, iib_ref, iou_ref,
                      out_ref, la, ab, conf, macc, negacc, coracc,
                      acc1, acc2, acc3):
    s = pl.program_id(0)
    i = pl.program_id(1)

    @pl.when((s == 0) & (i == 0))
    def _init():
        conf[...] = jnp.zeros_like(conf)
        macc[...] = jnp.zeros_like(macc)
        negacc[...] = jnp.zeros_like(negacc)
        coracc[...] = jnp.zeros_like(coracc)
        acc1[...] = jnp.zeros_like(acc1)
        acc2[...] = jnp.zeros_like(acc2)
        acc3[...] = jnp.zeros_like(acc3)

    @pl.when(s == 0)
    def _stage0():
        onesr = jnp.ones((1, _BN), jnp.float32)
        lb = logits_ref[...]
        bp = jnp.clip(jax.nn.sigmoid(lb), _SC, 1.0 - _SC)
        onem = 1.0 - bp
        l1m = jnp.log(onem)
        fneg = -l1m * bp * bp
        negacc[...] += lax.dot_general(
            onesr, fneg, _DN_STD, preferred_element_type=jnp.float32)
        iib = iib_ref[...]
        iibf = iib.astype(jnp.float32)
        mask = iib > 0
        kk = lax.dot_general(iibf, jnp.ones((_G, 1), jnp.float32), _DN_STD,
                             preferred_element_type=jnp.float32)
        b = -fneg[:, 0:1]
        logp0 = jnp.clip(lb[:, 0:1] + l1m[:, 0:1],
                         jnp.log(_SC), jnp.log(1.0 - _SC))
        a = logp0 * onem[:, 0:1] * onem[:, 0:1]
        selb = jnp.where(kk > 0.0, b, 0.0)
        coracc[...] += lax.dot_general(
            onesr, selb, _DN_STD, preferred_element_type=jnp.float32)
        ab2 = jnp.concatenate([a, b], axis=1)
        ab[pl.ds(i * _BN, _BN), :] = ab2
        acc3[...] += lax.dot_general(
            ab2, iibf, _DN_TT, preferred_element_type=jnp.float32)
        align = lax.dot_general(scores_ref[...], oh_ref[...], _DN_STD,
                                preferred_element_type=jnp.float32) * iibf
        conf[...] = jnp.maximum(conf[...],
                                jnp.max(align, axis=0, keepdims=True))
        la[pl.ds(i * _BN, _BN), :] = jnp.where(mask, align, -1.0)

    @pl.when(s == 1)
    def _stage1():
        giou = lax.dot_general(iou_ref[...], oh_ref[...], _DN_STD,
                               preferred_element_type=jnp.float32)
        lab = la[pl.ds(i * _BN, _BN), :]
        maskb = lab >= 0.0
        lg = jnp.log(jnp.maximum(lab, 1e-38))
        p = jnp.exp(conf[...] * lg)
        q = p * giou + _SC
        mq = jnp.where(maskb, q, 0.0)
        mq2 = mq * q
        macc[...] = jnp.maximum(macc[...],
                                jnp.max(mq, axis=0, keepdims=True))
        abb = ab[pl.ds(i * _BN, _BN), :]
        acc1[...] += lax.dot_general(
            abb, mq2, _DN_TT, preferred_element_type=jnp.float32)
        acc2[...] += lax.dot_general(
            abb, mq, _DN_TT, preferred_element_type=jnp.float32)

    @pl.when((s == 1) & (i == _NB - 1))
    def _final():
        rinv = 1.0 / jnp.maximum(macc[...], _SC)
        rinv2 = rinv * rinv
        posv = acc1[0:1, :] * rinv2
        bnegv = (acc3[1:2, :] - 2.0 * acc2[1:2, :] * rinv
                 + acc1[1:2, :] * rinv2)
        sneg = jnp.sum(negacc[...], axis=1, keepdims=True)
        spos = jnp.sum(posv, axis=1, keepdims=True)
        sbneg = jnp.sum(bnegv, axis=1, keepdims=True)
        total = ((1.0 - _ALPHA) * (sneg + coracc[...])
                 - _ALPHA * (spos + sbneg))
        out_ref[...] = jnp.broadcast_to(total, out_ref.shape)


def kernel(logits_pred, scores, iou_map, is_in_boxes, labels, num_pos_avg):
    oh = (labels[None, :] ==
          jax.lax.broadcasted_iota(jnp.int32, (_C, _G), 0)).astype(jnp.float32)
    out = pl.pallas_call(
        _mask_loss_kernel,
        grid=(2, _NB),
        in_specs=[
            pl.BlockSpec((_C, _G), lambda s, i: (0, 0)),
            pl.BlockSpec((_BN, _C),
                         lambda s, i: (jnp.where(s == 0, i, _NB - 1), 0)),
            pl.BlockSpec((_BN, _C),
                         lambda s, i: (jnp.where(s == 0, i, _NB - 1), 0)),
            pl.BlockSpec((_BN, _G),
                         lambda s, i: (jnp.where(s == 0, i, _NB - 1), 0)),
            pl.BlockSpec((_BN, _C), lambda s, i: (jnp.where(s == 1, i, 0), 0)),
        ],
        out_specs=pl.BlockSpec((8, 128), lambda s, i: (0, 0)),
        out_shape=jax.ShapeDtypeStruct((8, 128), jnp.float32),
        scratch_shapes=[
            pltpu.VMEM((_N, _G), jnp.float32),
            pltpu.VMEM((_N, 2), jnp.float32),
            pltpu.VMEM((1, _G), jnp.float32),
            pltpu.VMEM((1, _G), jnp.float32),
            pltpu.VMEM((1, _C), jnp.float32),
            pltpu.VMEM((1, 1), jnp.float32),
            pltpu.VMEM((2, _G), jnp.float32),
            pltpu.VMEM((2, _G), jnp.float32),
            pltpu.VMEM((2, _G), jnp.float32),
        ],
        compiler_params=pltpu.CompilerParams(
            dimension_semantics=("arbitrary", "arbitrary")),
    )(oh, logits_pred, scores, is_in_boxes, iou_map)
    return out[0, 0] / num_pos_avg


# speedup vs baseline: 4.1767x; 1.0293x over previous
import jax
import jax.numpy as jnp
from jax import lax
from jax.experimental import pallas as pl
from jax.experimental.pallas import tpu as pltpu

_GAMMA = 2.0
_SC = 0.0001
_ALPHA = 0.25

_N = 20000
_C = 80
_G = 128
_BN = 4000
_NB = _N // _BN

_DN_STD = (((1,), (0,)), ((), ()))
_DN_TT = (((0,), (0,)), ((), ()))


def _mask_loss_kernel(oh_ref, logits_ref, scores_ref, iib_ref, iou_ref,
                      npv_ref, out_ref, la, ab, conf, macc, negacc, coracc,
                      acc1, acc2, acc3):
    s = pl.program_id(0)
    i = pl.program_id(1)

    @pl.when((s == 0) & (i == 0))
    def _init():
        conf[...] = jnp.zeros_like(conf)
        macc[...] = jnp.zeros_like(macc)
        negacc[...] = jnp.zeros_like(negacc)
        coracc[...] = jnp.zeros_like(coracc)
        acc1[...] = jnp.zeros_like(acc1)
        acc2[...] = jnp.zeros_like(acc2)
        acc3[...] = jnp.zeros_like(acc3)

    @pl.when(s == 0)
    def _stage0():
        onesr = jnp.ones((1, _BN), jnp.float32)
        lb = logits_ref[...]
        bp = jnp.clip(jax.nn.sigmoid(lb), _SC, 1.0 - _SC)
        onem = 1.0 - bp
        l1m = jnp.log(onem)
        fneg = -l1m * bp * bp
        negacc[...] += lax.dot_general(
            onesr, fneg, _DN_STD, preferred_element_type=jnp.float32)
        iib = iib_ref[...]
        iibf = iib.astype(jnp.float32)
        mask = iib > 0
        kk = lax.dot_general(iibf, jnp.ones((_G, 1), jnp.float32), _DN_STD,
                             preferred_element_type=jnp.float32)
        b = -fneg[:, 0:1]
        logp0 = jnp.clip(lb[:, 0:1] + l1m[:, 0:1],
                         jnp.log(_SC), jnp.log(1.0 - _SC))
        a = logp0 * onem[:, 0:1] * onem[:, 0:1]
        selb = jnp.where(kk > 0.0, b, 0.0)
        coracc[...] += lax.dot_general(
            onesr, selb, _DN_STD, preferred_element_type=jnp.float32)
        ab2 = jnp.concatenate([a, b], axis=1)
        ab[pl.ds(i * _BN, _BN), :] = ab2
        acc3[...] += lax.dot_general(
            ab2, iibf, _DN_TT, preferred_element_type=jnp.float32)
        align = lax.dot_general(scores_ref[...], oh_ref[...], _DN_STD,
                                preferred_element_type=jnp.float32) * iibf
        conf[...] = jnp.maximum(conf[...],
                                jnp.max(align, axis=0, keepdims=True))
        la[pl.ds(i * _BN, _BN), :] = jnp.where(mask, align, -1.0)

    @pl.when(s == 1)
    def _stage1():
        giou = lax.dot_general(iou_ref[...], oh_ref[...], _DN_STD,
                               preferred_element_type=jnp.float32)
        lab = la[pl.ds(i * _BN, _BN), :]
        maskb = lab >= 0.0
        lg = jnp.log(jnp.maximum(lab, 1e-38))
        p = jnp.exp(conf[...] * lg)
        q = p * giou + _SC
        mq = jnp.where(maskb, q, 0.0)
        mq2 = mq * q
        macc[...] = jnp.maximum(macc[...],
                                jnp.max(mq, axis=0, keepdims=True))
        abb = ab[pl.ds(i * _BN, _BN), :]
        acc1[...] += lax.dot_general(
            abb, mq2, _DN_TT, preferred_element_type=jnp.float32)
        acc2[...] += lax.dot_general(
            abb, mq, _DN_TT, preferred_element_type=jnp.float32)

    @pl.when((s == 1) & (i == _NB - 1))
    def _final():
        rinv = 1.0 / jnp.maximum(macc[...], _SC)
        rinv2 = rinv * rinv
        posv = acc1[0:1, :] * rinv2
        bnegv = (acc3[1:2, :] - 2.0 * acc2[1:2, :] * rinv
                 + acc1[1:2, :] * rinv2)
        sneg = jnp.sum(negacc[...], axis=1, keepdims=True)
        spos = jnp.sum(posv, axis=1, keepdims=True)
        sbneg = jnp.sum(bnegv, axis=1, keepdims=True)
        total = ((1.0 - _ALPHA) * (sneg + coracc[...])
                 - _ALPHA * (spos + sbneg))
        out_ref[...] = total / npv_ref[...]


def kernel(logits_pred, scores, iou_map, is_in_boxes, labels, num_pos_avg):
    oh = (labels[None, :] ==
          jax.lax.broadcasted_iota(jnp.int32, (_C, _G), 0)).astype(jnp.float32)
    out = pl.pallas_call(
        _mask_loss_kernel,
        grid=(2, _NB),
        in_specs=[
            pl.BlockSpec((_C, _G), lambda s, i: (0, 0)),
            pl.BlockSpec((_BN, _C),
                         lambda s, i: (jnp.where(s == 0, i, _NB - 1), 0)),
            pl.BlockSpec((_BN, _C),
                         lambda s, i: (jnp.where(s == 0, i, _NB - 1), 0)),
            pl.BlockSpec((_BN, _G),
                         lambda s, i: (jnp.where(s == 0, i, _NB - 1), 0)),
            pl.BlockSpec((_BN, _C), lambda s, i: (jnp.where(s == 1, i, 0), 0)),
            pl.BlockSpec((1, 1), lambda s, i: (0, 0)),
        ],
        out_specs=pl.BlockSpec((1, 1), lambda s, i: (0, 0)),
        out_shape=jax.ShapeDtypeStruct((1, 1), jnp.float32),
        scratch_shapes=[
            pltpu.VMEM((_N, _G), jnp.float32),
            pltpu.VMEM((_N, 2), jnp.float32),
            pltpu.VMEM((1, _G), jnp.float32),
            pltpu.VMEM((1, _G), jnp.float32),
            pltpu.VMEM((1, _C), jnp.float32),
            pltpu.VMEM((1, 1), jnp.float32),
            pltpu.VMEM((2, _G), jnp.float32),
            pltpu.VMEM((2, _G), jnp.float32),
            pltpu.VMEM((2, _G), jnp.float32),
        ],
        compiler_params=pltpu.CompilerParams(
            dimension_semantics=("arbitrary", "arbitrary")),
    )(oh, logits_pred, scores, is_in_boxes, iou_map,
      jnp.asarray(num_pos_avg, jnp.float32).reshape(1, 1))
    return out[0, 0]


# speedup vs baseline: 4.2320x; 1.0132x over previous
import jax
import jax.numpy as jnp
from jax import lax
from jax.experimental import pallas as pl
from jax.experimental.pallas import tpu as pltpu

_GAMMA = 2.0
_SC = 0.0001
_ALPHA = 0.25

_N = 20000
_C = 80
_G = 128
_BN = 4000
_NB = _N // _BN

_DN_STD = (((1,), (0,)), ((), ()))
_DN_TT = (((0,), (0,)), ((), ()))


def _mask_loss_kernel(oh_ref, logits_ref, scores_ref, iib_ref, iou_ref,
                      npv_ref, out_ref, la, ab, conf, macc, negacc, coracc,
                      acc1, acc2, acc3):
    s = pl.program_id(0)
    i = pl.program_id(1)

    @pl.when((s == 0) & (i == 0))
    def _init():
        conf[...] = jnp.zeros_like(conf)
        macc[...] = jnp.zeros_like(macc)
        negacc[...] = jnp.zeros_like(negacc)
        coracc[...] = jnp.zeros_like(coracc)
        acc1[...] = jnp.zeros_like(acc1)
        acc2[...] = jnp.zeros_like(acc2)
        acc3[...] = jnp.zeros_like(acc3)

    @pl.when(s == 0)
    def _stage0():
        onesr = jnp.ones((1, _BN), jnp.float32)
        lb = logits_ref[...]
        bp = jnp.clip(jax.nn.sigmoid(lb), _SC, 1.0 - _SC)
        onem = 1.0 - bp
        l1m = jnp.log(onem)
        fneg = l1m * bp * bp
        negacc[...] += lax.dot_general(
            onesr, fneg, _DN_STD, preferred_element_type=jnp.float32)
        iib = iib_ref[...]
        iibf = iib.astype(jnp.float32)
        mask = iib > 0
        kk = lax.dot_general(iibf, jnp.ones((_G, 1), jnp.float32), _DN_STD,
                             preferred_element_type=jnp.float32)
        b = fneg[:, 0:1]
        logp0 = jnp.clip(lb[:, 0:1] + l1m[:, 0:1],
                         jnp.log(_SC), jnp.log(1.0 - _SC))
        a = logp0 * onem[:, 0:1] * onem[:, 0:1]
        selb = jnp.where(kk > 0.0, b, 0.0)
        coracc[...] += lax.dot_general(
            onesr, selb, _DN_STD, preferred_element_type=jnp.float32)
        ab2 = jnp.concatenate([a, b], axis=1)
        ab[pl.ds(i * _BN, _BN), :] = ab2
        acc3[...] += lax.dot_general(
            ab2, iibf, _DN_TT, preferred_element_type=jnp.float32)
        align = lax.dot_general(scores_ref[...], oh_ref[...], _DN_STD,
                                preferred_element_type=jnp.float32) * iibf
        conf[...] = jnp.maximum(conf[...],
                                jnp.max(align, axis=0, keepdims=True))
        la[pl.ds(i * _BN, _BN), :] = jnp.where(mask, align, -1.0)

    @pl.when(s == 1)
    def _stage1():
        giou = lax.dot_general(iou_ref[...], oh_ref[...], _DN_STD,
                               preferred_element_type=jnp.float32)
        lab = la[pl.ds(i * _BN, _BN), :]
        maskb = lab >= 0.0
        lg = jnp.log(jnp.maximum(lab, 1e-38))
        p = jnp.exp(conf[...] * lg)
        q = p * giou + _SC
        mq = jnp.where(maskb, q, 0.0)
        mq2 = mq * q
        macc[...] = jnp.maximum(macc[...],
                                jnp.max(mq, axis=0, keepdims=True))
        abb = ab[pl.ds(i * _BN, _BN), :]
        acc1[...] += lax.dot_general(
            abb, mq2, _DN_TT, preferred_element_type=jnp.float32)
        acc2[...] += lax.dot_general(
            abb, mq, _DN_TT, preferred_element_type=jnp.float32)

    @pl.when((s == 1) & (i == _NB - 1))
    def _final():
        rinv = 1.0 / jnp.maximum(macc[...], _SC)
        rinv2 = rinv * rinv
        posv = acc1[0:1, :] * rinv2
        bnegv = (acc3[1:2, :] - 2.0 * acc2[1:2, :] * rinv
                 + acc1[1:2, :] * rinv2)
        sneg = -jnp.sum(negacc[...], axis=1, keepdims=True)
        spos = jnp.sum(posv, axis=1, keepdims=True)
        sbneg = jnp.sum(bnegv, axis=1, keepdims=True)
        total = ((1.0 - _ALPHA) * (sneg + coracc[...])
                 - _ALPHA * (spos + sbneg))
        out_ref[...] = total / npv_ref[...]


def kernel(logits_pred, scores, iou_map, is_in_boxes, labels, num_pos_avg):
    oh = (labels[None, :] ==
          jax.lax.broadcasted_iota(jnp.int32, (_C, _G), 0)).astype(jnp.float32)
    out = pl.pallas_call(
        _mask_loss_kernel,
        grid=(2, _NB),
        in_specs=[
            pl.BlockSpec((_C, _G), lambda s, i: (0, 0)),
            pl.BlockSpec((_BN, _C),
                         lambda s, i: (jnp.where(s == 0, i, _NB - 1), 0)),
            pl.BlockSpec((_BN, _C),
                         lambda s, i: (jnp.where(s == 0, i, _NB - 1), 0)),
            pl.BlockSpec((_BN, _G),
                         lambda s, i: (jnp.where(s == 0, i, _NB - 1), 0)),
            pl.BlockSpec((_BN, _C), lambda s, i: (jnp.where(s == 1, i, 0), 0)),
            pl.BlockSpec((1, 1), lambda s, i: (0, 0)),
        ],
        out_specs=pl.BlockSpec((1, 1), lambda s, i: (0, 0)),
        out_shape=jax.ShapeDtypeStruct((1, 1), jnp.float32),
        scratch_shapes=[
            pltpu.VMEM((_N, _G), jnp.float32),
            pltpu.VMEM((_N, 2), jnp.float32),
            pltpu.VMEM((1, _G), jnp.float32),
            pltpu.VMEM((1, _G), jnp.float32),
            pltpu.VMEM((1, _C), jnp.float32),
            pltpu.VMEM((1, 1), jnp.float32),
            pltpu.VMEM((2, _G), jnp.float32),
            pltpu.VMEM((2, _G), jnp.float32),
            pltpu.VMEM((2, _G), jnp.float32),
        ],
        compiler_params=pltpu.CompilerParams(
            dimension_semantics=("arbitrary", "arbitrary")),
    )(oh, logits_pred, scores, is_in_boxes, iou_map,
      jnp.asarray(num_pos_avg, jnp.float32).reshape(1, 1))
    return out[0, 0]


# speedup vs baseline: 4.2522x; 1.0048x over previous
import jax
import jax.numpy as jnp
from jax import lax
from jax.experimental import pallas as pl
from jax.experimental.pallas import tpu as pltpu

_GAMMA = 2.0
_SC = 0.0001
_ALPHA = 0.25

_N = 20000
_C = 80
_G = 128
_BN = 4000
_NB = _N // _BN

_DN_STD = (((1,), (0,)), ((), ()))
_DN_TT = (((0,), (0,)), ((), ()))


def _mask_loss_kernel(oh_ref, logits_ref, scores_ref, iib_ref, iou_ref,
                      npv_ref, out_ref, la, ab, conf, macc, negacc, coracc,
                      acc1, acc2, acc3):
    s = pl.program_id(0)
    i = pl.program_id(1)

    @pl.when((s == 0) & (i == 0))
    def _init():
        conf[...] = jnp.zeros_like(conf)
        macc[...] = jnp.zeros_like(macc)
        negacc[...] = jnp.zeros_like(negacc)
        coracc[...] = jnp.zeros_like(coracc)
        acc1[...] = jnp.zeros_like(acc1)
        acc2[...] = jnp.zeros_like(acc2)
        acc3[...] = jnp.zeros_like(acc3)

    @pl.when(s == 0)
    def _stage0():
        onesr = jnp.ones((1, _BN), jnp.float32)
        lb = logits_ref[...]
        bp = jnp.clip(jax.nn.sigmoid(lb), _SC, 1.0 - _SC)
        onem = 1.0 - bp
        l1m = jnp.log(onem)
        fneg = l1m * bp * bp
        negacc[...] += lax.dot_general(
            onesr, fneg, _DN_STD, preferred_element_type=jnp.float32)
        iib = iib_ref[...]
        iibf = iib.astype(jnp.float32)
        mask = iib > 0
        kk = lax.dot_general(iibf, jnp.ones((_G, 1), jnp.float32), _DN_STD,
                             preferred_element_type=jnp.float32)
        b = fneg[:, 0:1]
        logp0 = jnp.clip(lb[:, 0:1] + l1m[:, 0:1],
                         jnp.log(_SC), jnp.log(1.0 - _SC))
        a = logp0 * onem[:, 0:1] * onem[:, 0:1]
        selb = jnp.where(kk > 0.0, b, 0.0)
        coracc[...] += lax.dot_general(
            onesr, selb, _DN_STD, preferred_element_type=jnp.float32)
        ab2 = jnp.concatenate([a, b], axis=1)
        ab[pl.ds(i * _BN, _BN), :] = ab2
        acc3[...] += lax.dot_general(
            ab2, iibf, _DN_TT, preferred_element_type=jnp.float32)
        align = lax.dot_general(scores_ref[...], oh_ref[...], _DN_STD,
                                preferred_element_type=jnp.float32) * iibf
        conf[...] = jnp.maximum(conf[...],
                                jnp.max(align, axis=0, keepdims=True))
        la[pl.ds(i * _BN, _BN), :] = jnp.where(
            mask, jnp.log(jnp.maximum(align, 1e-38)), 1.0)

    @pl.when(s == 1)
    def _stage1():
        giou = lax.dot_general(iou_ref[...], oh_ref[...], _DN_STD,
                               preferred_element_type=jnp.float32)
        lab = la[pl.ds(i * _BN, _BN), :]
        maskb = lab < 0.5
        p = jnp.exp(conf[...] * lab)
        q = p * giou + _SC
        mq = jnp.where(maskb, q, 0.0)
        mq2 = mq * q
        macc[...] = jnp.maximum(macc[...],
                                jnp.max(mq, axis=0, keepdims=True))
        abb = ab[pl.ds(i * _BN, _BN), :]
        acc1[...] += lax.dot_general(
            abb, mq2, _DN_TT, preferred_element_type=jnp.float32)
        acc2[...] += lax.dot_general(
            abb, mq, _DN_TT, preferred_element_type=jnp.float32)

    @pl.when((s == 1) & (i == _NB - 1))
    def _final():
        rinv = 1.0 / jnp.maximum(macc[...], _SC)
        rinv2 = rinv * rinv
        posv = acc1[0:1, :] * rinv2
        bnegv = (acc3[1:2, :] - 2.0 * acc2[1:2, :] * rinv
                 + acc1[1:2, :] * rinv2)
        sneg = -jnp.sum(negacc[...], axis=1, keepdims=True)
        spos = jnp.sum(posv, axis=1, keepdims=True)
        sbneg = jnp.sum(bnegv, axis=1, keepdims=True)
        total = ((1.0 - _ALPHA) * (sneg + coracc[...])
                 - _ALPHA * (spos + sbneg))
        out_ref[...] = total / npv_ref[...]


def kernel(logits_pred, scores, iou_map, is_in_boxes, labels, num_pos_avg):
    oh = (labels[None, :] ==
          jax.lax.broadcasted_iota(jnp.int32, (_C, _G), 0)).astype(jnp.float32)
    out = pl.pallas_call(
        _mask_loss_kernel,
        grid=(2, _NB),
        in_specs=[
            pl.BlockSpec((_C, _G), lambda s, i: (0, 0)),
            pl.BlockSpec((_BN, _C),
                         lambda s, i: (jnp.where(s == 0, i, _NB - 1), 0)),
            pl.BlockSpec((_BN, _C),
                         lambda s, i: (jnp.where(s == 0, i, _NB - 1), 0)),
            pl.BlockSpec((_BN, _G),
                         lambda s, i: (jnp.where(s == 0, i, _NB - 1), 0)),
            pl.BlockSpec((_BN, _C), lambda s, i: (jnp.where(s == 1, i, 0), 0)),
            pl.BlockSpec((1, 1), lambda s, i: (0, 0)),
        ],
        out_specs=pl.BlockSpec((1, 1), lambda s, i: (0, 0)),
        out_shape=jax.ShapeDtypeStruct((1, 1), jnp.float32),
        scratch_shapes=[
            pltpu.VMEM((_N, _G), jnp.float32),
            pltpu.VMEM((_N, 2), jnp.float32),
            pltpu.VMEM((1, _G), jnp.float32),
            pltpu.VMEM((1, _G), jnp.float32),
            pltpu.VMEM((1, _C), jnp.float32),
            pltpu.VMEM((1, 1), jnp.float32),
            pltpu.VMEM((2, _G), jnp.float32),
            pltpu.VMEM((2, _G), jnp.float32),
            pltpu.VMEM((2, _G), jnp.float32),
        ],
        compiler_params=pltpu.CompilerParams(
            dimension_semantics=("arbitrary", "arbitrary")),
    )(oh, logits_pred, scores, is_in_boxes, iou_map,
      jnp.asarray(num_pos_avg, jnp.float32).reshape(1, 1))
    return out[0, 0]


# speedup vs baseline: 4.4203x; 1.0395x over previous
import jax
import jax.numpy as jnp
from jax import lax
from jax.experimental import pallas as pl
from jax.experimental.pallas import tpu as pltpu

_GAMMA = 2.0
_SC = 0.0001
_ALPHA = 0.25

_N = 20000
_C = 80
_G = 128
_BN = 4000
_NB = _N // _BN

_DN_STD = (((1,), (0,)), ((), ()))
_DN_TT = (((0,), (0,)), ((), ()))


def _mask_loss_kernel(lab_ref, logits_ref, scores_ref, iib_ref, iou_ref,
                      npv_ref, out_ref, la, ab, conf, macc, negacc, coracc,
                      acc1, acc2, acc3):
    s = pl.program_id(0)
    i = pl.program_id(1)

    @pl.when((s == 0) & (i == 0))
    def _init():
        conf[...] = jnp.zeros_like(conf)
        macc[...] = jnp.zeros_like(macc)
        negacc[...] = jnp.zeros_like(negacc)
        coracc[...] = jnp.zeros_like(coracc)
        acc1[...] = jnp.zeros_like(acc1)
        acc2[...] = jnp.zeros_like(acc2)
        acc3[...] = jnp.zeros_like(acc3)

    @pl.when(s == 0)
    def _stage0():
        onesr = jnp.ones((1, _BN), jnp.float32)
        lb = logits_ref[...]
        bp = jnp.clip(jax.nn.sigmoid(lb), _SC, 1.0 - _SC)
        onem = 1.0 - bp
        l1m = jnp.log(onem)
        fneg = l1m * bp * bp
        negacc[...] += lax.dot_general(
            onesr, fneg, _DN_STD, preferred_element_type=jnp.float32)
        iib = iib_ref[...]
        iibf = iib.astype(jnp.float32)
        mask = iib > 0
        kk = lax.dot_general(iibf, jnp.ones((_G, 1), jnp.float32), _DN_STD,
                             preferred_element_type=jnp.float32)
        b = fneg[:, 0:1]
        logp0 = jnp.clip(lb[:, 0:1] + l1m[:, 0:1],
                         jnp.log(_SC), jnp.log(1.0 - _SC))
        a = logp0 * onem[:, 0:1] * onem[:, 0:1]
        selb = jnp.where(kk > 0.0, b, 0.0)
        coracc[...] += lax.dot_general(
            onesr, selb, _DN_STD, preferred_element_type=jnp.float32)
        ab2 = jnp.concatenate([a, b], axis=1)
        ab[pl.ds(i * _BN, _BN), :] = ab2
        acc3[...] += lax.dot_general(
            ab2, iibf, _DN_TT, preferred_element_type=jnp.float32)
        oh = (lab_ref[...] ==
              jax.lax.broadcasted_iota(jnp.int32, (_C, _G), 0)
              ).astype(jnp.float32)
        align = lax.dot_general(scores_ref[...], oh, _DN_STD,
                                preferred_element_type=jnp.float32) * iibf
        conf[...] = jnp.maximum(conf[...],
                                jnp.max(align, axis=0, keepdims=True))
        la[pl.ds(i * _BN, _BN), :] = jnp.where(
            mask, jnp.log(jnp.maximum(align, 1e-38)), 1.0)

    @pl.when(s == 1)
    def _stage1():
        oh = (lab_ref[...] ==
              jax.lax.broadcasted_iota(jnp.int32, (_C, _G), 0)
              ).astype(jnp.float32)
        giou = lax.dot_general(iou_ref[...], oh, _DN_STD,
                               preferred_element_type=jnp.float32)
        lab = la[pl.ds(i * _BN, _BN), :]
        maskb = lab < 0.5
        p = jnp.exp(conf[...] * lab)
        q = p * giou + _SC
        mq = jnp.where(maskb, q, 0.0)
        mq2 = mq * q
        macc[...] = jnp.maximum(macc[...],
                                jnp.max(mq, axis=0, keepdims=True))
        abb = ab[pl.ds(i * _BN, _BN), :]
        acc1[...] += lax.dot_general(
            abb, mq2, _DN_TT, preferred_element_type=jnp.float32)
        acc2[...] += lax.dot_general(
            abb, mq, _DN_TT, preferred_element_type=jnp.float32)

    @pl.when((s == 1) & (i == _NB - 1))
    def _final():
        rinv = 1.0 / jnp.maximum(macc[...], _SC)
        rinv2 = rinv * rinv
        posv = acc1[0:1, :] * rinv2
        bnegv = (acc3[1:2, :] - 2.0 * acc2[1:2, :] * rinv
                 + acc1[1:2, :] * rinv2)
        sneg = -jnp.sum(negacc[...], axis=1, keepdims=True)
        spos = jnp.sum(posv, axis=1, keepdims=True)
        sbneg = jnp.sum(bnegv, axis=1, keepdims=True)
        total = ((1.0 - _ALPHA) * (sneg + coracc[...])
                 - _ALPHA * (spos + sbneg))
        out_ref[...] = total / npv_ref[...]


def kernel(logits_pred, scores, iou_map, is_in_boxes, labels, num_pos_avg):
    out = pl.pallas_call(
        _mask_loss_kernel,
        grid=(2, _NB),
        in_specs=[
            pl.BlockSpec((1, _G), lambda s, i: (0, 0)),
            pl.BlockSpec((_BN, _C),
                         lambda s, i: (jnp.where(s == 0, i, _NB - 1), 0)),
            pl.BlockSpec((_BN, _C),
                         lambda s, i: (jnp.where(s == 0, i, _NB - 1), 0)),
            pl.BlockSpec((_BN, _G),
                         lambda s, i: (jnp.where(s == 0, i, _NB - 1), 0)),
            pl.BlockSpec((_BN, _C), lambda s, i: (jnp.where(s == 1, i, 0), 0)),
            pl.BlockSpec((1, 1), lambda s, i: (0, 0)),
        ],
        out_specs=pl.BlockSpec((1, 1), lambda s, i: (0, 0)),
        out_shape=jax.ShapeDtypeStruct((1, 1), jnp.float32),
        scratch_shapes=[
            pltpu.VMEM((_N, _G), jnp.float32),
            pltpu.VMEM((_N, 2), jnp.float32),
            pltpu.VMEM((1, _G), jnp.float32),
            pltpu.VMEM((1, _G), jnp.float32),
            pltpu.VMEM((1, _C), jnp.float32),
            pltpu.VMEM((1, 1), jnp.float32),
            pltpu.VMEM((2, _G), jnp.float32),
            pltpu.VMEM((2, _G), jnp.float32),
            pltpu.VMEM((2, _G), jnp.float32),
        ],
        compiler_params=pltpu.CompilerParams(
            dimension_semantics=("arbitrary", "arbitrary")),
    )(labels[None, :], logits_pred, scores, is_in_boxes, iou_map,
      jnp.asarray(num_pos_avg, jnp.float32).reshape(1, 1))
    return out[0, 0]


# speedup vs baseline: 4.4528x; 1.0073x over previous
import jax
import jax.numpy as jnp
from jax import lax
from jax.experimental import pallas as pl
from jax.experimental.pallas import tpu as pltpu

_GAMMA = 2.0
_SC = 0.0001
_ALPHA = 0.25

_N = 20000
_C = 80
_G = 128
_BN = 4000
_NB = _N // _BN

_DN_STD = (((1,), (0,)), ((), ()))
_DN_TT = (((0,), (0,)), ((), ()))


def _mask_loss_kernel(lab_ref, logits_ref, scores_ref, iib_ref, iou_ref,
                      npv_ref, out_ref, la, ab, conf, macc, negacc, coracc,
                      acc1, acc2, acc3):
    s = pl.program_id(0)
    i = pl.program_id(1)

    @pl.when((s == 0) & (i == 0))
    def _init():
        conf[...] = jnp.zeros_like(conf)
        macc[...] = jnp.zeros_like(macc)
        negacc[...] = jnp.zeros_like(negacc)
        coracc[...] = jnp.zeros_like(coracc)
        acc1[...] = jnp.zeros_like(acc1)
        acc2[...] = jnp.zeros_like(acc2)
        acc3[...] = jnp.zeros_like(acc3)

    @pl.when(s == 0)
    def _stage0():
        onesr = jnp.ones((1, _BN), jnp.float32)
        lb = logits_ref[...]
        bp = jnp.clip(jax.nn.sigmoid(lb), _SC, 1.0 - _SC)
        onem = 1.0 - bp
        l1m = jnp.log(onem)
        fneg = l1m * bp * bp
        negacc[...] += lax.dot_general(
            onesr, fneg, _DN_STD, preferred_element_type=jnp.float32)
        iib = iib_ref[...]
        iibf = iib.astype(jnp.float32)
        mask = iib > 0
        kk = lax.dot_general(iibf, jnp.ones((_G, 1), jnp.float32), _DN_STD,
                             preferred_element_type=jnp.float32)
        b = fneg[:, 0:1]
        logp0 = jnp.clip(lb[:, 0:1] + l1m[:, 0:1],
                         jnp.log(_SC), jnp.log(1.0 - _SC))
        a = logp0 * onem[:, 0:1] * onem[:, 0:1]
        selb = jnp.where(kk > 0.0, b, 0.0)
        coracc[...] += lax.dot_general(
            onesr, selb, _DN_STD, preferred_element_type=jnp.float32)
        ab2 = jnp.concatenate([a, b], axis=1)
        ab[pl.ds(i * _BN, _BN), :] = ab2
        acc3[...] += lax.dot_general(
            ab2, iibf, _DN_TT, preferred_element_type=jnp.float32)
        oh = (lab_ref[...] ==
              jax.lax.broadcasted_iota(jnp.int32, (_C, _G), 0)
              ).astype(jnp.float32)
        align = lax.dot_general(scores_ref[...], oh, _DN_STD,
                                preferred_element_type=jnp.float32) * iibf
        conf[...] = jnp.maximum(conf[...],
                                jnp.max(align, axis=0, keepdims=True))
        la[pl.ds(i * _BN, _BN), :] = jnp.where(
            mask, jnp.log(jnp.maximum(align, 1e-38)), -1e9)

    @pl.when(s == 1)
    def _stage1():
        oh = (lab_ref[...] ==
              jax.lax.broadcasted_iota(jnp.int32, (_C, _G), 0)
              ).astype(jnp.float32)
        giou = lax.dot_general(iou_ref[...], oh, _DN_STD,
                               preferred_element_type=jnp.float32)
        lab = la[pl.ds(i * _BN, _BN), :]
        p = jnp.exp(conf[...] * lab)
        mpw = p * giou
        mpw2 = mpw * mpw
        macc[...] = jnp.maximum(macc[...],
                                jnp.max(mpw, axis=0, keepdims=True))
        abb = ab[pl.ds(i * _BN, _BN), :]
        acc1[...] += lax.dot_general(
            abb, mpw2, _DN_TT, preferred_element_type=jnp.float32)
        acc2[...] += lax.dot_general(
            abb, mpw, _DN_TT, preferred_element_type=jnp.float32)

    @pl.when((s == 1) & (i == _NB - 1))
    def _final():
        rinv = 1.0 / (macc[...] + _SC)
        rinv2 = rinv * rinv
        a2q = acc1[0:1, :] + 2.0 * _SC * acc2[0:1, :] + _SC * _SC * acc3[0:1, :]
        k1q = acc2[1:2, :] + _SC * acc3[1:2, :]
        k2q = acc1[1:2, :] + 2.0 * _SC * acc2[1:2, :] + _SC * _SC * acc3[1:2, :]
        posv = a2q * rinv2
        bnegv = acc3[1:2, :] - 2.0 * k1q * rinv + k2q * rinv2
        sneg = -jnp.sum(negacc[...], axis=1, keepdims=True)
        spos = jnp.sum(posv, axis=1, keepdims=True)
        sbneg = jnp.sum(bnegv, axis=1, keepdims=True)
        total = ((1.0 - _ALPHA) * (sneg + coracc[...])
                 - _ALPHA * (spos + sbneg))
        out_ref[...] = total / npv_ref[...]


def kernel(logits_pred, scores, iou_map, is_in_boxes, labels, num_pos_avg):
    out = pl.pallas_call(
        _mask_loss_kernel,
        grid=(2, _NB),
        in_specs=[
            pl.BlockSpec((1, _G), lambda s, i: (0, 0)),
            pl.BlockSpec((_BN, _C),
                         lambda s, i: (jnp.where(s == 0, i, _NB - 1), 0)),
            pl.BlockSpec((_BN, _C),
                         lambda s, i: (jnp.where(s == 0, i, _NB - 1), 0)),
            pl.BlockSpec((_BN, _G),
                         lambda s, i: (jnp.where(s == 0, i, _NB - 1), 0)),
            pl.BlockSpec((_BN, _C), lambda s, i: (jnp.where(s == 1, i, 0), 0)),
            pl.BlockSpec((1, 1), lambda s, i: (0, 0)),
        ],
        out_specs=pl.BlockSpec((1, 1), lambda s, i: (0, 0)),
        out_shape=jax.ShapeDtypeStruct((1, 1), jnp.float32),
        scratch_shapes=[
            pltpu.VMEM((_N, _G), jnp.float32),
            pltpu.VMEM((_N, 2), jnp.float32),
            pltpu.VMEM((1, _G), jnp.float32),
            pltpu.VMEM((1, _G), jnp.float32),
            pltpu.VMEM((1, _C), jnp.float32),
            pltpu.VMEM((1, 1), jnp.float32),
            pltpu.VMEM((2, _G), jnp.float32),
            pltpu.VMEM((2, _G), jnp.float32),
            pltpu.VMEM((2, _G), jnp.float32),
        ],
        compiler_params=pltpu.CompilerParams(
            dimension_semantics=("arbitrary", "arbitrary")),
    )(labels[None, :], logits_pred, scores, is_in_boxes, iou_map,
      jnp.asarray(num_pos_avg, jnp.float32).reshape(1, 1))
    return out[0, 0]
